# baseline (speedup 1.0000x reference)
import jax
import jax.numpy as jnp
from jax.experimental import pallas as pl
from jax.experimental.pallas import tpu as pltpu

N = 10000
BI = 400
BQ = 1000
F8 = jnp.float8_e4m3fn


def _stats_accumulate(sn, cs_ref, am_ref):
    cs = jnp.sum(sn, axis=0, keepdims=True)
    am = jnp.max(jnp.abs(sn), axis=0, keepdims=True)
    i = pl.program_id(0)

    @pl.when(i == 0)
    def _():
        cs_ref[...] = cs
        am_ref[...] = am

    @pl.when(i > 0)
    def _():
        cs_ref[...] = cs_ref[...] + cs
        am_ref[...] = jnp.maximum(am_ref[...], am)


def _next_support(h, w_ref, snext_ref, cs_ref, am_ref):
    sn = jnp.dot(h, w_ref[...], preferred_element_type=jnp.float32)
    snext_ref[...] = sn.astype(jnp.bfloat16)
    _stats_accumulate(sn, cs_ref, am_ref)


def _layer1_body(adj_ref, x_ref, w1_ref, b_ref, w_ref,
                 snext_ref, cs_ref, am_ref, qadj_ref, s1_scr):
    @pl.when(pl.program_id(0) == 0)
    def _():
        s1_scr[...] = jnp.dot(x_ref[...], w1_ref[...],
                              preferred_element_type=jnp.float32
                              ).astype(jnp.bfloat16)

    a32 = adj_ref[...]
    qadj_ref[...] = (a32 - 0.5).astype(F8)
    acc = jnp.dot(a32.astype(jnp.bfloat16), s1_scr[...],
                  preferred_element_type=jnp.float32)
    h = jnp.maximum(acc + b_ref[...], 0.0)
    _next_support(h, w_ref, snext_ref, cs_ref, am_ref)


def _quant_to_scratch(s_ref, sigma, qs_scr):
    @pl.when(pl.program_id(0) == 0)
    def _():
        qs_scr[...] = (s_ref[...].astype(jnp.float32) / sigma).astype(F8)


def _dequant_acc(qa_ref, qs_scr, sigma, cs_ref, b_ref):
    acc = jnp.dot(qa_ref[...], qs_scr[...],
                  preferred_element_type=jnp.float32)
    return acc * sigma + 0.5 * cs_ref[...] + b_ref[...]


def _qlayer_body(qa_ref, s_ref, am_ref, cs_ref, b_ref, w_ref,
                 snext_ref, cs2_ref, am2_ref, qs_scr):
    sigma = jnp.maximum(jnp.max(am_ref[...]), 1e-20) / 240.0
    _quant_to_scratch(s_ref, sigma, qs_scr)
    h = jnp.maximum(_dequant_acc(qa_ref, qs_scr, sigma, cs_ref, b_ref), 0.0)
    _next_support(h, w_ref, snext_ref, cs2_ref, am2_ref)


def _qfinal_body(qa_ref, s_ref, am_ref, cs_ref, b_ref, o_ref, qs_scr):
    sigma = jnp.maximum(jnp.max(am_ref[...]), 1e-20) / 240.0
    _quant_to_scratch(s_ref, sigma, qs_scr)
    o_ref[...] = _dequant_acc(qa_ref, qs_scr, sigma, cs_ref, b_ref)


def _layer1(adj, x, w1, b, w_next):
    fi = w1.shape[1]
    fo = w_next.shape[1]
    return pl.pallas_call(
        _layer1_body,
        grid=(N // BI,),
        in_specs=[
            pl.BlockSpec((BI, N), lambda i: (i, 0)),
            pl.BlockSpec((N, x.shape[1]), lambda i: (0, 0)),
            pl.BlockSpec((x.shape[1], fi), lambda i: (0, 0)),
            pl.BlockSpec((1, fi), lambda i: (0, 0)),
            pl.BlockSpec((fi, fo), lambda i: (0, 0)),
        ],
        out_specs=[
            pl.BlockSpec((BI, fo), lambda i: (i, 0)),
            pl.BlockSpec((1, fo), lambda i: (0, 0)),
            pl.BlockSpec((1, fo), lambda i: (0, 0)),
            pl.BlockSpec((BI, N), lambda i: (i, 0)),
        ],
        out_shape=[
            jax.ShapeDtypeStruct((N, fo), jnp.bfloat16),
            jax.ShapeDtypeStruct((1, fo), jnp.float32),
            jax.ShapeDtypeStruct((1, fo), jnp.float32),
            jax.ShapeDtypeStruct((N, N), F8),
        ],
        scratch_shapes=[pltpu.VMEM((N, fi), jnp.bfloat16)],
    )(adj, x, w1, b.reshape(1, fi), w_next)


def _qlayer(qadj, s, am, cs, b, w_next):
    fi = s.shape[1]
    fo = w_next.shape[1]
    return pl.pallas_call(
        _qlayer_body,
        grid=(N // BQ,),
        in_specs=[
            pl.BlockSpec((BQ, N), lambda i: (i, 0)),
            pl.BlockSpec((N, fi), lambda i: (0, 0)),
            pl.BlockSpec((1, fi), lambda i: (0, 0)),
            pl.BlockSpec((1, fi), lambda i: (0, 0)),
            pl.BlockSpec((1, fi), lambda i: (0, 0)),
            pl.BlockSpec((fi, fo), lambda i: (0, 0)),
        ],
        out_specs=[
            pl.BlockSpec((BQ, fo), lambda i: (i, 0)),
            pl.BlockSpec((1, fo), lambda i: (0, 0)),
            pl.BlockSpec((1, fo), lambda i: (0, 0)),
        ],
        out_shape=[
            jax.ShapeDtypeStruct((N, fo), jnp.bfloat16),
            jax.ShapeDtypeStruct((1, fo), jnp.float32),
            jax.ShapeDtypeStruct((1, fo), jnp.float32),
        ],
        scratch_shapes=[pltpu.VMEM((N, fi), F8)],
    )(qadj, s, am, cs, b.reshape(1, fi), w_next)


def _qfinal(qadj, s, am, cs, b):
    fo = s.shape[1]
    return pl.pallas_call(
        _qfinal_body,
        grid=(N // BQ,),
        in_specs=[
            pl.BlockSpec((BQ, N), lambda i: (i, 0)),
            pl.BlockSpec((N, fo), lambda i: (0, 0)),
            pl.BlockSpec((1, fo), lambda i: (0, 0)),
            pl.BlockSpec((1, fo), lambda i: (0, 0)),
            pl.BlockSpec((1, fo), lambda i: (0, 0)),
        ],
        out_specs=pl.BlockSpec((BQ, fo), lambda i: (i, 0)),
        out_shape=jax.ShapeDtypeStruct((N, fo), jnp.float32),
        scratch_shapes=[pltpu.VMEM((N, fo), F8)],
    )(qadj, s, am, cs, b.reshape(1, fo))


def kernel(x, adj, W1, b1, W2, b2, W3, b3, W4, b4, W5, b5, W6, b6):
    s, cs, am, qadj = _layer1(adj, x, W1, b1, W2)
    for b, w in ((b2, W3), (b3, W4), (b4, W5), (b5, W6)):
        s, cs, am = _qlayer(qadj, s, am, cs, b, w)
    return _qfinal(qadj, s, am, cs, b6)


# speedup vs baseline: 1.0314x; 1.0314x over previous
import jax
import jax.numpy as jnp
from jax.experimental import pallas as pl
from jax.experimental.pallas import tpu as pltpu

N = 10000
BI = 400
BQ = 1000
NL = 5
F8 = jnp.float8_e4m3fn


def _stats_accumulate(sn, cs_ref, am_ref, i):
    cs = jnp.sum(sn, axis=0, keepdims=True)
    am = jnp.max(jnp.abs(sn), axis=0, keepdims=True)

    @pl.when(i == 0)
    def _():
        cs_ref[...] = cs
        am_ref[...] = am

    @pl.when(i > 0)
    def _():
        cs_ref[...] = cs_ref[...] + cs
        am_ref[...] = jnp.maximum(am_ref[...], am)


def _layer1_body(adj_ref, x_ref, w1_ref, b_ref, w_ref,
                 snext_ref, cs_ref, am_ref, qadj_ref, s1_scr):
    i = pl.program_id(0)

    @pl.when(i == 0)
    def _():
        s1_scr[...] = jnp.dot(x_ref[...], w1_ref[...],
                              preferred_element_type=jnp.float32
                              ).astype(jnp.bfloat16)

    a32 = adj_ref[...]
    qadj_ref[...] = (a32 - 0.5).astype(F8)
    acc = jnp.dot(a32.astype(jnp.bfloat16), s1_scr[...],
                  preferred_element_type=jnp.float32)
    h = jnp.maximum(acc + b_ref[...], 0.0)
    sn = jnp.dot(h, w_ref[...], preferred_element_type=jnp.float32)
    snext_ref[...] = sn.astype(jnp.bfloat16)
    _stats_accumulate(sn, cs_ref, am_ref, i)


def _mega_body(qadj_ref, sin_ref, csin_ref, amin_ref, bs_ref, ws_ref,
               out_ref, s_a, s_b, qs_scr, cs_cur, am_cur, cs_nxt, am_nxt):
    l = pl.program_id(0)
    i = pl.program_id(1)

    @pl.when((l == 0) & (i == 0))
    def _():
        s_a[...] = sin_ref[...].astype(jnp.float32)
        cs_cur[...] = csin_ref[...]
        am_cur[...] = amin_ref[...]

    @pl.when((l > 0) & (i == 0))
    def _():
        cs_cur[...] = cs_nxt[...]
        am_cur[...] = am_nxt[...]

    sigma = jnp.maximum(jnp.max(am_cur[...]), 1e-20) / 240.0

    @pl.when(i == 0)
    def _():
        src = jnp.where((l % 2) == 0, s_a[...], s_b[...])
        qs_scr[...] = (src / sigma).astype(F8)

    acc = jnp.dot(qadj_ref[...], qs_scr[...],
                  preferred_element_type=jnp.float32)
    pre = acc * sigma + 0.5 * cs_cur[...] + bs_ref[0]
    h = jnp.maximum(pre, 0.0)
    sn = jnp.dot(h, ws_ref[0], preferred_element_type=jnp.float32)

    @pl.when((l < NL - 1) & (l % 2 == 0))
    def _():
        s_b[pl.ds(i * BQ, BQ), :] = sn

    @pl.when((l < NL - 1) & (l % 2 == 1))
    def _():
        s_a[pl.ds(i * BQ, BQ), :] = sn

    @pl.when(l < NL - 1)
    def _():
        _stats_accumulate(sn, cs_nxt, am_nxt, i)

    out_ref[...] = pre


def _layer1(adj, x, w1, b, w_next):
    fi = w1.shape[1]
    fo = w_next.shape[1]
    return pl.pallas_call(
        _layer1_body,
        grid=(N // BI,),
        in_specs=[
            pl.BlockSpec((BI, N), lambda i: (i, 0)),
            pl.BlockSpec((N, x.shape[1]), lambda i: (0, 0)),
            pl.BlockSpec((x.shape[1], fi), lambda i: (0, 0)),
            pl.BlockSpec((1, fi), lambda i: (0, 0)),
            pl.BlockSpec((fi, fo), lambda i: (0, 0)),
        ],
        out_specs=[
            pl.BlockSpec((BI, fo), lambda i: (i, 0)),
            pl.BlockSpec((1, fo), lambda i: (0, 0)),
            pl.BlockSpec((1, fo), lambda i: (0, 0)),
            pl.BlockSpec((BI, N), lambda i: (i, 0)),
        ],
        out_shape=[
            jax.ShapeDtypeStruct((N, fo), jnp.bfloat16),
            jax.ShapeDtypeStruct((1, fo), jnp.float32),
            jax.ShapeDtypeStruct((1, fo), jnp.float32),
            jax.ShapeDtypeStruct((N, N), F8),
        ],
        scratch_shapes=[pltpu.VMEM((N, fi), jnp.bfloat16)],
    )(adj, x, w1, b.reshape(1, fi), w_next)


def _mega(qadj, s, cs, am, bs, ws):
    f = s.shape[1]
    return pl.pallas_call(
        _mega_body,
        grid=(NL, N // BQ),
        in_specs=[
            pl.BlockSpec((BQ, N), lambda l, i: (i, 0)),
            pl.BlockSpec((N, f), lambda l, i: (0, 0)),
            pl.BlockSpec((1, f), lambda l, i: (0, 0)),
            pl.BlockSpec((1, f), lambda l, i: (0, 0)),
            pl.BlockSpec((1, 1, f), lambda l, i: (l, 0, 0)),
            pl.BlockSpec((1, f, f), lambda l, i: (jnp.minimum(l, NL - 2), 0, 0)),
        ],
        out_specs=pl.BlockSpec(
            (BQ, f), lambda l, i: (jnp.where(l == NL - 1, i, 0), 0)),
        out_shape=jax.ShapeDtypeStruct((N, f), jnp.float32),
        scratch_shapes=[
            pltpu.VMEM((N, f), jnp.float32),
            pltpu.VMEM((N, f), jnp.float32),
            pltpu.VMEM((N, f), F8),
            pltpu.VMEM((1, f), jnp.float32),
            pltpu.VMEM((1, f), jnp.float32),
            pltpu.VMEM((1, f), jnp.float32),
            pltpu.VMEM((1, f), jnp.float32),
        ],
    )(qadj, s, cs, am, bs, ws)


def kernel(x, adj, W1, b1, W2, b2, W3, b3, W4, b4, W5, b5, W6, b6):
    s, cs, am, qadj = _layer1(adj, x, W1, b1, W2)
    f = s.shape[1]
    nc = W6.shape[1]
    w6p = jnp.pad(W6, ((0, 0), (0, f - nc)))
    b6p = jnp.pad(b6, (0, f - nc))
    bs = jnp.stack([b2, b3, b4, b5, b6p]).reshape(NL, 1, f)
    ws = jnp.stack([W3, W4, W5, w6p])
    out = _mega(qadj, s, cs, am, bs, ws)
    return out[:, :nc]


# speedup vs baseline: 1.0428x; 1.0110x over previous
import jax
import jax.numpy as jnp
from jax.experimental import pallas as pl
from jax.experimental.pallas import tpu as pltpu

N = 10000
BI = 400
BQ = 1000
NL = 5
F8 = jnp.float8_e4m3fn


def _stats_accumulate(sn, cs_ref, am_ref, i):
    cs = jnp.sum(sn, axis=0, keepdims=True)
    am = jnp.max(jnp.abs(sn), axis=0, keepdims=True)

    @pl.when(i == 0)
    def _():
        cs_ref[...] = cs
        am_ref[...] = am

    @pl.when(i > 0)
    def _():
        cs_ref[...] = cs_ref[...] + cs
        am_ref[...] = jnp.maximum(am_ref[...], am)


def _layer1_body(adj_ref, x_ref, w1_ref, b_ref, w_ref,
                 snext_ref, cs_ref, am_ref, qadj_ref, s1_scr):
    i = pl.program_id(0)

    @pl.when(i == 0)
    def _():
        s1_scr[...] = jnp.dot(x_ref[...], w1_ref[...],
                              preferred_element_type=jnp.float32
                              ).astype(jnp.bfloat16)

    a32 = adj_ref[...]
    qadj_ref[...] = (a32 - 0.5).astype(F8)
    acc = jnp.dot(a32.astype(jnp.bfloat16), s1_scr[...],
                  preferred_element_type=jnp.float32)
    h = jnp.maximum(acc + b_ref[...], 0.0)
    sn = jnp.dot(h, w_ref[...], preferred_element_type=jnp.float32)
    snext_ref[...] = sn.astype(jnp.bfloat16)
    _stats_accumulate(sn, cs_ref, am_ref, i)


def _mega_body(qadj_ref, sin_ref, csin_ref, amin_ref, bs_ref, ws_ref,
               out_ref, s_a, s_b, qs_scr, cs_cur, am_cur, cs_nxt, am_nxt):
    l = pl.program_id(0)
    i = pl.program_id(1)

    @pl.when((l == 0) & (i == 0))
    def _():
        s_a[...] = sin_ref[...].astype(jnp.float32)
        cs_cur[...] = csin_ref[...]
        am_cur[...] = amin_ref[...]

    @pl.when((l > 0) & (i == 0))
    def _():
        cs_cur[...] = cs_nxt[...]
        am_cur[...] = am_nxt[...]

    sigma = jnp.maximum(jnp.max(am_cur[...]), 1e-20) / 240.0

    @pl.when(i == 0)
    def _():
        src = jnp.where((l % 2) == 0, s_a[...], s_b[...])
        qs_scr[...] = (src / sigma).astype(F8)

    acc = jnp.dot(qadj_ref[...], qs_scr[...],
                  preferred_element_type=jnp.float32)
    pre = acc * sigma + 0.5 * cs_cur[...] + bs_ref[0]

    @pl.when(l < NL - 1)
    def _():
        h = jnp.maximum(pre, 0.0)
        sn = jnp.dot(h, ws_ref[0], preferred_element_type=jnp.float32)

        @pl.when(l % 2 == 0)
        def _():
            s_b[pl.ds(i * BQ, BQ), :] = sn

        @pl.when(l % 2 == 1)
        def _():
            s_a[pl.ds(i * BQ, BQ), :] = sn

        _stats_accumulate(sn, cs_nxt, am_nxt, i)

    @pl.when(l == NL - 1)
    def _():
        out_ref[...] = pre


def _layer1(adj, x, w1, b, w_next):
    fi = w1.shape[1]
    fo = w_next.shape[1]
    return pl.pallas_call(
        _layer1_body,
        grid=(N // BI,),
        in_specs=[
            pl.BlockSpec((BI, N), lambda i: (i, 0)),
            pl.BlockSpec((N, x.shape[1]), lambda i: (0, 0)),
            pl.BlockSpec((x.shape[1], fi), lambda i: (0, 0)),
            pl.BlockSpec((1, fi), lambda i: (0, 0)),
            pl.BlockSpec((fi, fo), lambda i: (0, 0)),
        ],
        out_specs=[
            pl.BlockSpec((BI, fo), lambda i: (i, 0)),
            pl.BlockSpec((1, fo), lambda i: (0, 0)),
            pl.BlockSpec((1, fo), lambda i: (0, 0)),
            pl.BlockSpec((BI, N), lambda i: (i, 0)),
        ],
        out_shape=[
            jax.ShapeDtypeStruct((N, fo), jnp.bfloat16),
            jax.ShapeDtypeStruct((1, fo), jnp.float32),
            jax.ShapeDtypeStruct((1, fo), jnp.float32),
            jax.ShapeDtypeStruct((N, N), F8),
        ],
        scratch_shapes=[pltpu.VMEM((N, fi), jnp.bfloat16)],
    )(adj, x, w1, b.reshape(1, fi), w_next)


def _mega(qadj, s, cs, am, bs, ws):
    f = s.shape[1]
    return pl.pallas_call(
        _mega_body,
        grid=(NL, N // BQ),
        in_specs=[
            pl.BlockSpec((BQ, N), lambda l, i: (i, 0)),
            pl.BlockSpec((N, f), lambda l, i: (0, 0)),
            pl.BlockSpec((1, f), lambda l, i: (0, 0)),
            pl.BlockSpec((1, f), lambda l, i: (0, 0)),
            pl.BlockSpec((1, 1, f), lambda l, i: (l, 0, 0)),
            pl.BlockSpec((1, f, f), lambda l, i: (jnp.minimum(l, NL - 2), 0, 0)),
        ],
        out_specs=pl.BlockSpec(
            (BQ, f), lambda l, i: (jnp.where(l == NL - 1, i, 0), 0)),
        out_shape=jax.ShapeDtypeStruct((N, f), jnp.float32),
        scratch_shapes=[
            pltpu.VMEM((N, f), jnp.float32),
            pltpu.VMEM((N, f), jnp.float32),
            pltpu.VMEM((N, f), F8),
            pltpu.VMEM((1, f), jnp.float32),
            pltpu.VMEM((1, f), jnp.float32),
            pltpu.VMEM((1, f), jnp.float32),
            pltpu.VMEM((1, f), jnp.float32),
        ],
    )(qadj, s, cs, am, bs, ws)


def kernel(x, adj, W1, b1, W2, b2, W3, b3, W4, b4, W5, b5, W6, b6):
    s, cs, am, qadj = _layer1(adj, x, W1, b1, W2)
    f = s.shape[1]
    nc = W6.shape[1]
    w6p = jnp.pad(W6, ((0, 0), (0, f - nc)))
    b6p = jnp.pad(b6, (0, f - nc))
    bs = jnp.stack([b2, b3, b4, b5, b6p]).reshape(NL, 1, f)
    ws = jnp.stack([W3, W4, W5, w6p])
    out = _mega(qadj, s, cs, am, bs, ws)
    return out[:, :nc]
